# initial kernel scaffold (unmeasured)
import jax
import jax.numpy as jnp
from jax import lax
from jax.experimental import pallas as pl
from jax.experimental.pallas import tpu as pltpu

N_DEV = 16
BM = 512


def kernel(x, w_mat):
    m, k_per = x.shape
    k_tot, n = w_mat.shape

    def body(x_ref, w_ref, out_ref, xb_ref, comm_ref, send_sems, recv_sems):
        k = pl.program_id(0)
        me = lax.axis_index("i")

        @pl.when(k == 0)
        def _first():
            xb_ref[...] = x_ref[...].astype(jnp.bfloat16)
            out_ref[...] = jnp.zeros_like(out_ref)
            for d in range(N_DEV):
                @pl.when(d != me)
                def _send(d=d):
                    rdma = pltpu.make_async_remote_copy(
                        src_ref=xb_ref.at[pl.ds(d * BM, BM), :],
                        dst_ref=comm_ref.at[me],
                        send_sem=send_sems.at[d],
                        recv_sem=recv_sems.at[me],
                        device_id=(d,),
                        device_id_type=pl.DeviceIdType.MESH,
                    )
                    rdma.start()

        @pl.when(k != me)
        def _wait():
            recv = pltpu.make_async_remote_copy(
                src_ref=comm_ref.at[k],
                dst_ref=comm_ref.at[k],
                send_sem=send_sems.at[k],
                recv_sem=recv_sems.at[k],
                device_id=(me,),
                device_id_type=pl.DeviceIdType.MESH,
            )
            recv.wait_recv()

        a_own = xb_ref[pl.ds(me * BM, BM), :]
        a = jnp.where(k == me, a_own, comm_ref[k])
        wb = w_ref[...].astype(jnp.bfloat16)
        out_ref[...] += jnp.dot(a, wb, preferred_element_type=jnp.float32)

        @pl.when(k == N_DEV - 1)
        def _fin():
            for d in range(N_DEV):
                @pl.when(d != me)
                def _wait_send(d=d):
                    s = pltpu.make_async_remote_copy(
                        src_ref=xb_ref.at[pl.ds(d * BM, BM), :],
                        dst_ref=comm_ref.at[me],
                        send_sem=send_sems.at[d],
                        recv_sem=recv_sems.at[me],
                        device_id=(d,),
                        device_id_type=pl.DeviceIdType.MESH,
                    )
                    s.wait_send()
            y = out_ref[...]
            out_ref[...] = y * (1.0 / (1.0 + jnp.exp(-y)))

    return pl.pallas_call(
        body,
        grid=(N_DEV,),
        in_specs=[
            pl.BlockSpec((m, k_per), lambda k: (0, 0)),
            pl.BlockSpec((BM, n), lambda k: (k, 0)),
        ],
        out_specs=pl.BlockSpec((BM, n), lambda k: (0, 0)),
        out_shape=jax.ShapeDtypeStruct((BM, n), jnp.float32),
        scratch_shapes=[
            pltpu.VMEM((m, k_per), jnp.bfloat16),
            pltpu.VMEM((N_DEV, BM, k_per), jnp.bfloat16),
            pltpu.SemaphoreType.DMA((N_DEV,)),
            pltpu.SemaphoreType.DMA((N_DEV,)),
        ],
        compiler_params=pltpu.CompilerParams(
            dimension_semantics=("arbitrary",),
        ),
    )(x, w_mat)


# baseline (device time: 128535 ns/iter reference)
import jax
import jax.numpy as jnp
from jax import lax
from jax.experimental import pallas as pl
from jax.experimental.pallas import tpu as pltpu

N_DEV = 16
BM = 512
CHUNK = 2048
BLKS_PER_CHUNK = CHUNK // BM


def kernel(x, w_mat):
    m, k_per = x.shape
    k_tot, n = w_mat.shape
    n_chunks = m // CHUNK

    def body(x_hbm, w_ref, out_ref, xf_ref, xb_ref, comm_ref,
             xsems, send_sems, recv_sems):
        k = pl.program_id(0)
        me = lax.axis_index("i")

        def send_block(d):
            @pl.when(d != me)
            def _send(d=d):
                rdma = pltpu.make_async_remote_copy(
                    src_ref=xb_ref.at[pl.ds(d * BM, BM), :],
                    dst_ref=comm_ref.at[me],
                    send_sem=send_sems.at[d],
                    recv_sem=recv_sems.at[me],
                    device_id=(d,),
                    device_id_type=pl.DeviceIdType.MESH,
                )
                rdma.start()

        @pl.when(k == 0)
        def _first():
            out_ref[...] = jnp.zeros_like(out_ref)
            pltpu.make_async_copy(
                x_hbm.at[pl.ds(0, CHUNK), :], xf_ref.at[0], xsems.at[0]
            ).start()
            for c in range(n_chunks):
                if c + 1 < n_chunks:
                    pltpu.make_async_copy(
                        x_hbm.at[pl.ds((c + 1) * CHUNK, CHUNK), :],
                        xf_ref.at[(c + 1) % 2],
                        xsems.at[(c + 1) % 2],
                    ).start()
                pltpu.make_async_copy(
                    x_hbm.at[pl.ds(c * CHUNK, CHUNK), :],
                    xf_ref.at[c % 2],
                    xsems.at[c % 2],
                ).wait()
                xb_ref[pl.ds(c * CHUNK, CHUNK), :] = (
                    xf_ref[c % 2].astype(jnp.bfloat16)
                )
                for dd in range(BLKS_PER_CHUNK):
                    send_block(c * BLKS_PER_CHUNK + dd)

        @pl.when(k != me)
        def _wait():
            recv = pltpu.make_async_remote_copy(
                src_ref=comm_ref.at[k],
                dst_ref=comm_ref.at[k],
                send_sem=send_sems.at[k],
                recv_sem=recv_sems.at[k],
                device_id=(me,),
                device_id_type=pl.DeviceIdType.MESH,
            )
            recv.wait_recv()

        a_own = xb_ref[pl.ds(me * BM, BM), :]
        a = jnp.where(k == me, a_own, comm_ref[k])
        wb = w_ref[...].astype(jnp.bfloat16)
        out_ref[...] += jnp.dot(a, wb, preferred_element_type=jnp.float32)

        @pl.when(k == N_DEV - 1)
        def _fin():
            for d in range(N_DEV):
                @pl.when(d != me)
                def _wait_send(d=d):
                    s = pltpu.make_async_remote_copy(
                        src_ref=xb_ref.at[pl.ds(d * BM, BM), :],
                        dst_ref=comm_ref.at[me],
                        send_sem=send_sems.at[d],
                        recv_sem=recv_sems.at[me],
                        device_id=(d,),
                        device_id_type=pl.DeviceIdType.MESH,
                    )
                    s.wait_send()
            y = out_ref[...]
            out_ref[...] = y * (1.0 / (1.0 + jnp.exp(-y)))

    return pl.pallas_call(
        body,
        grid=(N_DEV,),
        in_specs=[
            pl.BlockSpec(memory_space=pl.ANY),
            pl.BlockSpec((BM, n), lambda k: (k, 0)),
        ],
        out_specs=pl.BlockSpec((BM, n), lambda k: (0, 0)),
        out_shape=jax.ShapeDtypeStruct((BM, n), jnp.float32),
        scratch_shapes=[
            pltpu.VMEM((2, CHUNK, k_per), jnp.float32),
            pltpu.VMEM((m, k_per), jnp.bfloat16),
            pltpu.VMEM((N_DEV, BM, k_per), jnp.bfloat16),
            pltpu.SemaphoreType.DMA((2,)),
            pltpu.SemaphoreType.DMA((N_DEV,)),
            pltpu.SemaphoreType.DMA((N_DEV,)),
        ],
        compiler_params=pltpu.CompilerParams(
            dimension_semantics=("arbitrary",),
            vmem_limit_bytes=56 * 1024 * 1024,
        ),
    )(x, w_mat)
